# Initial kernel scaffold; baseline (speedup 1.0000x reference)
import jax
import jax.numpy as jnp
from jax.experimental import pallas as pl


def kernel(x, img_weight):
    raise NotImplementedError("write your pallas kernel here")


# speedup vs baseline: 1.8759x; 1.8759x over previous
import functools

import jax
import jax.numpy as jnp
from jax import lax
from jax.experimental import pallas as pl
from jax.experimental.pallas import tpu as pltpu
from jax.experimental.pallas import tpu_sc as plsc

_BATCH, _HIST, _D = 16384, 50, 64
_B = _BATCH * _HIST
_NC, _NS = 2, 16
_NW = _NC * _NS
_BPW = _B // _NW
_G = 128
_NG = _BPW // _G
_K = 4
_NOUT = _NG // _K
_NBUF = 2


def _body(xw_hbm, tab_hbm, out_hbm, idx_v, rows_v, gsems, ssems):
    wid = lax.axis_index("s") * _NC + lax.axis_index("c")
    base = wid * _BPW

    pltpu.sync_copy(xw_hbm.at[wid], idx_v)

    def fire(b, g):
        for q in range(_K):
            pltpu.async_copy(
                tab_hbm.at[idx_v.at[g * _K + q]],
                rows_v.at[b, pl.ds(q * _G, _G)],
                gsems.at[b],
            )

    def drain(b):
        pltpu.make_async_copy(
            tab_hbm.at[pl.ds(0, _K * _G)], rows_v.at[b], gsems.at[b]
        ).wait()

    def store(b, g):
        pltpu.async_copy(
            rows_v.at[b],
            out_hbm.at[pl.ds(base + g * _K * _G, _K * _G)],
            ssems.at[b],
        )

    def wait_store(b):
        pltpu.make_async_copy(
            rows_v.at[b], out_hbm.at[pl.ds(base, _K * _G)], ssems.at[b]
        ).wait()

    fire(0, 0)

    @pl.loop(0, _NOUT, step=_NBUF)
    def _outer(g0):
        for b in range(_NBUF):
            g = g0 + b
            nb = (b + 1) % _NBUF
            ng = g + 1

            @pl.when(ng < _NOUT)
            def _():
                @pl.when(ng >= _NBUF)
                def _():
                    wait_store(nb)

                fire(nb, ng)

            drain(b)
            store(b, g)

    for b in range(_NBUF):
        wait_store(b)


@jax.jit
def _lookup(x_flat, img_weight):
    mesh = plsc.VectorSubcoreMesh(core_axis_name="c", subcore_axis_name="s")
    run = functools.partial(
        pl.kernel,
        out_type=jax.ShapeDtypeStruct((_B, _D), jnp.float32),
        mesh=mesh,
        scratch_types=[
            pltpu.VMEM((_NG, _G), jnp.int32),
            pltpu.VMEM((_NBUF, _K * _G, _D), jnp.float32),
            pltpu.SemaphoreType.DMA((_NBUF,)),
            pltpu.SemaphoreType.DMA((_NBUF,)),
        ],
        compiler_params=pltpu.CompilerParams(use_tc_tiling_on_sc=False),
    )(_body)
    return run(x_flat, img_weight)


def kernel(x, img_weight):
    x_flat = x.reshape(_NW, _NG, _G)
    out = _lookup(x_flat, img_weight)
    return out.reshape(_BATCH, _HIST, _D)
